# baseline (speedup 1.0000x reference)
import functools

import jax
import jax.numpy as jnp
from jax import lax
from jax.experimental import pallas as pl
from jax.experimental.pallas import tpu as pltpu
from jax.experimental.pallas import tpu_sc as plsc

N = 10000
E = 160000
SEQ = 6
DIM = 2
LATENT = 128
STEPS = 10
TYPES = 9
R = 0.015

NC = 2
NS = 16
NW = NC * NS
C = 100
K = 5
KS = 5
CS = 50

NCH_G = 2 * E // C
CPW_G = NCH_G // NW
NCH_A = E // C
CPW_A = NCH_A // NW
NCH_S = E // CS
CPW_S = NCH_S // NW

NPAD = 10240
RPS = NPAD // NS

BN = 2000
BE = 2000
EB = E // BE


@functools.cache
def _mesh():
  return plsc.VectorSubcoreMesh(core_axis_name="c", subcore_axis_name="s",
                                num_cores=NC, num_subcores=NS)


@functools.cache
def _make_gather():
  @functools.partial(
      pl.kernel,
      mesh=_mesh(),
      out_type=jax.ShapeDtypeStruct((NCH_G, C, LATENT), jnp.float32),
      scratch_types=[pltpu.VMEM((CPW_G, C), jnp.int32)]
      + [pltpu.VMEM((C, LATENT), jnp.float32)] * K
      + [pltpu.SemaphoreType.DMA] * (K + 1),
  )
  def gather(tab, iall, out, idx_v, *rest):
    bufs = rest[:K]
    sems = rest[K:2 * K]
    wsem = rest[2 * K]
    wid = lax.axis_index("s") * NC + lax.axis_index("c")
    pltpu.sync_copy(iall.at[wid], idx_v)

    def rnd(r, carry):
      base = r * K
      hs = [pltpu.async_copy(tab.at[idx_v.at[base + b]], bufs[b], sems[b])
            for b in range(K)]
      ws = []
      for b in range(K):
        hs[b].wait()
        ws.append(pltpu.async_copy(
            bufs[b], out.at[wid * CPW_G + base + b], wsem))
      for w in ws:
        w.wait()
      return carry

    lax.fori_loop(0, CPW_G // K, rnd, 0)

  return gather


@functools.cache
def _make_gather_sum():
  @functools.partial(
      pl.kernel,
      mesh=_mesh(),
      out_type=jax.ShapeDtypeStruct((NCH_A, C, LATENT), jnp.float32),
      scratch_types=[pltpu.VMEM((CPW_A, C), jnp.int32),
                     pltpu.VMEM((CPW_A, C), jnp.int32)]
      + [pltpu.VMEM((C, LATENT), jnp.float32)] * K
      + [pltpu.SemaphoreType.DMA] * (K + 1),
  )
  def gather_sum(tab, ialls, iallr, out, ixs, ixr, *rest):
    bufs = rest[:K]
    sems = rest[K:2 * K]
    wsem = rest[2 * K]
    wid = lax.axis_index("s") * NC + lax.axis_index("c")
    pltpu.sync_copy(ialls.at[wid], ixs)
    pltpu.sync_copy(iallr.at[wid], ixr)

    def rnd(r, carry):
      base = r * K
      hs = [pltpu.async_copy(tab.at[ixs.at[base + b]], bufs[b], sems[b])
            for b in range(K)]
      ha = []
      for b in range(K):
        hs[b].wait()
        ha.append(pltpu.async_copy(tab.at[ixr.at[base + b]], bufs[b],
                                   sems[b], add=True))
      ws = []
      for b in range(K):
        ha[b].wait()
        ws.append(pltpu.async_copy(
            bufs[b], out.at[wid * CPW_A + base + b], wsem))
      for w in ws:
        w.wait()
      return carry

    lax.fori_loop(0, CPW_A // K, rnd, 0)

  return gather_sum


@functools.cache
def _make_scatter_add():
  @functools.partial(
      pl.kernel,
      mesh=_mesh(),
      out_type=jax.ShapeDtypeStruct((NC, NPAD, LATENT), jnp.float32),
      scratch_types=[pltpu.VMEM((CPW_S, CS), jnp.int32)]
      + [pltpu.VMEM((CS, LATENT), jnp.float32)] * KS
      + [pltpu.VMEM_SHARED((NPAD, LATENT), jnp.float32)]
      + [pltpu.SemaphoreType.DMA] * (KS + 1),
  )
  def scatter(el3, ircv, zinit, out, idx_v, *rest):
    bufs = rest[:KS]
    shared = rest[KS]
    sems = rest[KS + 1:2 * KS + 1]
    ssem = rest[2 * KS + 1]
    cid = lax.axis_index("c")
    sid = lax.axis_index("s")
    wid = sid * NC + cid
    pltpu.sync_copy(ircv.at[wid], idx_v)
    pltpu.sync_copy(zinit, shared.at[pl.ds(sid * RPS, RPS)])
    plsc.subcore_barrier()

    def rnd(r, carry):
      base = r * KS
      hs = [pltpu.async_copy(el3.at[wid * CPW_S + base + b], bufs[b], sems[b])
            for b in range(KS)]
      ss = []
      for b in range(KS):
        hs[b].wait()
        ss.append(pltpu.async_copy(
            bufs[b], shared.at[idx_v.at[base + b]], ssem, add=True))
      for s in ss:
        s.wait()
      return carry

    lax.fori_loop(0, CPW_S // KS, rnd, 0)
    plsc.subcore_barrier()
    pltpu.sync_copy(shared.at[pl.ds(sid * RPS, RPS)],
                    out.at[cid, pl.ds(sid * RPS, RPS)])

  return scatter


def _ln(y, g, b):
  m = jnp.mean(y, axis=-1, keepdims=True)
  v = jnp.mean((y - m) ** 2, axis=-1, keepdims=True)
  return (y - m) / jnp.sqrt(v + 1e-5) * g + b


def _dot(a, b):
  return jnp.dot(a, b, preferred_element_type=jnp.float32)


def _rows(bs, cols):
  return pl.BlockSpec((bs, cols), lambda i: (i, 0))


def _rows_off(bs, cols, off):
  return pl.BlockSpec((bs, cols), lambda i: (i + off, 0))


def _rep(shape):
  return pl.BlockSpec(shape, lambda i: (0,) * len(shape))


def _node_enc_body(cp, ty, temb, w1, b1, w2, b2, w3, b3, g, be, ws, wr,
                   nl_o, t_o):
  x = cp[...]
  vel = x[:, 2:12] - x[:, 0:10]
  mr = x[:, 10:12]
  dists = jnp.clip(
      jnp.concatenate([mr - 0.1, 0.9 - mr], axis=1) / R, -1.0, 1.0)
  iot = lax.broadcasted_iota(jnp.int32, (BN, 16), 1).astype(jnp.float32)
  oh = (ty[...] == iot).astype(jnp.float32)
  te = _dot(oh, temb[...])
  feat = jnp.concatenate(
      [vel, dists, te, jnp.zeros((BN, 2), jnp.float32)], axis=1)
  h = jnp.maximum(_dot(feat, w1[...]) + b1[...], 0.0)
  h = jnp.maximum(_dot(h, w2[...]) + b2[...], 0.0)
  nl = _ln(_dot(h, w3[...]) + b3[...], g[...], be[...])
  nl_o[...] = nl
  t_o[0] = _dot(nl, ws[...])
  t_o[1] = _dot(nl, wr[...])


def _edge0_body(gsum, ps, pr, ew1, eb1, ew2, eb2, ew3, eb3, eg, ebe,
                w1e, b1, w2, b2, w3, b3, g, be, el_o):
  rel = (ps[:, 0:2] - pr[:, 0:2]) / R
  rd = jnp.sqrt(jnp.sum(rel * rel, axis=1, keepdims=True))
  ef = jnp.concatenate([rel, rd, jnp.zeros((BE, 5), jnp.float32)], axis=1)
  h = jnp.maximum(_dot(ef, ew1[...]) + eb1[...], 0.0)
  h = jnp.maximum(_dot(h, ew2[...]) + eb2[...], 0.0)
  el = _ln(_dot(h, ew3[...]) + eb3[...], eg[...], ebe[...])
  h1 = jnp.maximum(_dot(el, w1e[...]) + gsum[...] + b1[...], 0.0)
  h2 = jnp.maximum(_dot(h1, w2[...]) + b2[...], 0.0)
  eu = _ln(_dot(h2, w3[...]) + b3[...], g[...], be[...])
  el_o[...] = el + eu


def _edge_body(el, gsum, w1e, b1, w2, b2, w3, b3, g, be, el_o):
  elv = el[...]
  h1 = jnp.maximum(_dot(elv, w1e[...]) + gsum[...] + b1[...], 0.0)
  h2 = jnp.maximum(_dot(h1, w2[...]) + b2[...], 0.0)
  eu = _ln(_dot(h2, w3[...]) + b3[...], g[...], be[...])
  el_o[...] = elv + eu


def _node_body(nl, p0, p1, v1n, v1a, b1, v2, b2, v3, b3, g, be, ws, wr,
               nl_o, t_o):
  nlv = nl[...]
  agg = p0[0] + p1[0]
  h1 = jnp.maximum(_dot(nlv, v1n[...]) + _dot(agg, v1a[...]) + b1[...], 0.0)
  h2 = jnp.maximum(_dot(h1, v2[...]) + b2[...], 0.0)
  nu = _ln(_dot(h2, v3[...]) + b3[...], g[...], be[...])
  nl2 = nlv + nu
  nl_o[...] = nl2
  t_o[0] = _dot(nl2, ws[...])
  t_o[1] = _dot(nl2, wr[...])


def _node_last_body(nl, p0, p1, v1n, v1a, b1, v2, b2, v3, b3, g, be, nl_o):
  nlv = nl[...]
  agg = p0[0] + p1[0]
  h1 = jnp.maximum(_dot(nlv, v1n[...]) + _dot(agg, v1a[...]) + b1[...], 0.0)
  h2 = jnp.maximum(_dot(h1, v2[...]) + b2[...], 0.0)
  nu = _ln(_dot(h2, v3[...]) + b3[...], g[...], be[...])
  nl_o[...] = nlv + nu


def _dec_body(cp, nl, d1, db1, d2, db2, d3, db3, out_o):
  x = cp[...]
  h = jnp.maximum(_dot(nl[...], d1[...]) + db1[...], 0.0)
  h = jnp.maximum(_dot(h, d2[...]) + db2[...], 0.0)
  acc = _dot(h, d3[...]) + db3[...]
  mr = x[:, 10:12]
  prev = x[:, 8:10]
  newpos = mr + (mr - prev) + acc[:, 0:2]
  out_o[...] = jnp.concatenate(
      [newpos, jnp.zeros((BN, 6), jnp.float32)], axis=1)


_W128 = _rep((LATENT, LATENT))
_B128 = _rep((1, LATENT))
_F32 = jax.ShapeDtypeStruct

_TSPEC = pl.BlockSpec((2, BN, LATENT), lambda i: (0, i, 0))
_TSHAPE = _F32((2, N, LATENT), jnp.float32)


def _node_enc_call(cp, ty, temb, args):
  return pl.pallas_call(
      _node_enc_body,
      grid=(N // BN,),
      in_specs=[_rows(BN, 12), _rows(BN, 1), _rep((16, 16)),
                _rep((32, LATENT)), _B128, _W128, _B128, _W128, _B128,
                _B128, _B128, _W128, _W128],
      out_specs=[_rows(BN, LATENT), _TSPEC],
      out_shape=[_F32((N, LATENT), jnp.float32), _TSHAPE],
  )(cp, ty, temb, *args)


def _edge0_call(gsum, pall, enc_args, st_args):
  return pl.pallas_call(
      _edge0_body,
      grid=(EB,),
      in_specs=[_rows(BE, LATENT),
                _rows(BE, LATENT), _rows_off(BE, LATENT, EB),
                _rep((8, LATENT)), _B128, _W128, _B128, _W128, _B128,
                _B128, _B128,
                _W128, _B128, _W128, _B128, _W128, _B128, _B128, _B128],
      out_specs=_rows(BE, LATENT),
      out_shape=_F32((E, LATENT), jnp.float32),
  )(gsum, pall, pall, *enc_args, *st_args)


def _edge_call(el, gsum, st_args):
  return pl.pallas_call(
      _edge_body,
      grid=(EB,),
      in_specs=[_rows(BE, LATENT), _rows(BE, LATENT),
                _W128, _B128, _W128, _B128, _W128, _B128, _B128, _B128],
      out_specs=_rows(BE, LATENT),
      out_shape=_F32((E, LATENT), jnp.float32),
  )(el, gsum, *st_args)


def _node_call(nl, p0, p1, st_args, ws, wr):
  return pl.pallas_call(
      _node_body,
      grid=(N // BN,),
      in_specs=[_rows(BN, LATENT),
                pl.BlockSpec((1, BN, LATENT), lambda i: (0, i, 0)),
                pl.BlockSpec((1, BN, LATENT), lambda i: (1, i, 0)),
                _W128, _W128, _B128, _W128, _B128, _W128, _B128, _B128,
                _B128, _W128, _W128],
      out_specs=[_rows(BN, LATENT), _TSPEC],
      out_shape=[_F32((N, LATENT), jnp.float32), _TSHAPE],
  )(nl, p0, p1, *st_args, ws, wr)


def _node_last_call(nl, p0, p1, st_args):
  return pl.pallas_call(
      _node_last_body,
      grid=(N // BN,),
      in_specs=[_rows(BN, LATENT),
                pl.BlockSpec((1, BN, LATENT), lambda i: (0, i, 0)),
                pl.BlockSpec((1, BN, LATENT), lambda i: (1, i, 0)),
                _W128, _W128, _B128, _W128, _B128, _W128, _B128, _B128,
                _B128],
      out_specs=_rows(BN, LATENT),
      out_shape=_F32((N, LATENT), jnp.float32),
  )(nl, p0, p1, *st_args)


def _dec_call(cp, nl, args):
  return pl.pallas_call(
      _dec_body,
      grid=(N // BN,),
      in_specs=[_rows(BN, 12), _rows(BN, LATENT),
                _W128, _B128, _W128, _B128, _rep((LATENT, 8)), _rep((1, 8))],
      out_specs=_rows(BN, 8),
      out_shape=_F32((N, 8), jnp.float32),
  )(cp, nl, *args)


def _b(x):
  return x.reshape(1, -1)


def kernel(current_positions, particle_types, edge_index, params):
  cp = current_positions.reshape(N, SEQ * DIM)
  ty = particle_types.astype(jnp.float32).reshape(N, 1)
  snd = edge_index[0].astype(jnp.int32)
  rcv = edge_index[1].astype(jnp.int32)
  iall = jnp.concatenate([snd, rcv + N]).reshape(NW, CPW_G, C)
  ialls = snd.reshape(NW, CPW_A, C)
  iallr = (rcv + N).reshape(NW, CPW_A, C)
  ircv3 = rcv.reshape(NW, CPW_S, CS)
  zinit = jnp.zeros((RPS, LATENT), jnp.float32)

  temb = jnp.pad(params["type_emb"], ((0, 16 - TYPES), (0, 0)))
  ne = params["node_enc"]
  ne_w1 = jnp.pad(ne["mlp"][0]["W"], ((0, 2), (0, 0)))
  ne_args = (ne_w1, _b(ne["mlp"][0]["b"]), ne["mlp"][1]["W"],
             _b(ne["mlp"][1]["b"]), ne["mlp"][2]["W"], _b(ne["mlp"][2]["b"]),
             _b(ne["g"]), _b(ne["be"]))
  ee = params["edge_enc"]
  ee_w1 = jnp.pad(ee["mlp"][0]["W"], ((0, 5), (0, 0)))
  ee_args = (ee_w1, _b(ee["mlp"][0]["b"]), ee["mlp"][1]["W"],
             _b(ee["mlp"][1]["b"]), ee["mlp"][2]["W"], _b(ee["mlp"][2]["b"]),
             _b(ee["g"]), _b(ee["be"]))

  esplit, eargs, nargs = [], [], []
  for st in params["proc"]:
    ew = st["edge"]["mlp"][0]["W"]
    esplit.append((ew[:LATENT], ew[LATENT:2 * LATENT], ew[2 * LATENT:]))
    em = st["edge"]
    eargs.append((em["mlp"][0]["W"][:LATENT], _b(em["mlp"][0]["b"]),
                  em["mlp"][1]["W"], _b(em["mlp"][1]["b"]),
                  em["mlp"][2]["W"], _b(em["mlp"][2]["b"]),
                  _b(em["g"]), _b(em["be"])))
    nm = st["node"]
    nw = nm["mlp"][0]["W"]
    nargs.append((nw[:LATENT], nw[LATENT:], _b(nm["mlp"][0]["b"]),
                  nm["mlp"][1]["W"], _b(nm["mlp"][1]["b"]),
                  nm["mlp"][2]["W"], _b(nm["mlp"][2]["b"]),
                  _b(nm["g"]), _b(nm["be"])))

  dm = params["dec"]["mlp"]
  d3 = jnp.pad(dm[2]["W"], ((0, 0), (0, 8 - DIM)))
  db3 = _b(jnp.pad(dm[2]["b"], (0, 8 - DIM)))
  dec_args = (dm[0]["W"], _b(dm[0]["b"]), dm[1]["W"], _b(dm[1]["b"]),
              d3, db3)

  nl, tstack = _node_enc_call(
      cp, ty, temb, ne_args + (esplit[0][1], esplit[0][2]))
  pos_tab = jnp.pad(cp[:, 10:12], ((0, 0), (0, LATENT - DIM)))
  pos2 = jnp.concatenate([pos_tab, pos_tab], axis=0)
  gsum = _make_gather_sum()(tstack.reshape(2 * N, LATENT), ialls, iallr)
  pall = _make_gather()(pos2, iall)
  el = _edge0_call(gsum.reshape(E, LATENT), pall.reshape(2 * E, LATENT),
                   ee_args, eargs[0])

  for t in range(STEPS):
    if t > 0:
      gsum = _make_gather_sum()(tstack.reshape(2 * N, LATENT), ialls, iallr)
      el = _edge_call(el, gsum.reshape(E, LATENT), eargs[t])
    parts = _make_scatter_add()(el.reshape(NCH_S, CS, LATENT), ircv3, zinit)
    if t < STEPS - 1:
      nl, tstack = _node_call(nl, parts, parts, nargs[t],
                              esplit[t + 1][1], esplit[t + 1][2])
    else:
      nl = _node_last_call(nl, parts, parts, nargs[t])

  out = _dec_call(cp, nl, dec_args)
  return out[:, :DIM]


# speedup vs baseline: 1.0211x; 1.0211x over previous
import functools

import jax
import jax.numpy as jnp
from jax import lax
from jax.experimental import pallas as pl
from jax.experimental.pallas import tpu as pltpu
from jax.experimental.pallas import tpu_sc as plsc

N = 10000
E = 160000
SEQ = 6
DIM = 2
LATENT = 128
STEPS = 10
TYPES = 9
R = 0.015

NC = 2
NS = 16
NW = NC * NS
C = 100
K = 5
KS = 5
CS = 50

NCH_A = E // C
CPW_A = NCH_A // NW
NCH_S = E // CS
CPW_S = NCH_S // NW

NPAD = 10240
RPS = NPAD // NS

BN = 2000
BE = 2000
EB = E // BE


@functools.cache
def _mesh():
  return plsc.VectorSubcoreMesh(core_axis_name="c", subcore_axis_name="s",
                                num_cores=NC, num_subcores=NS)


@functools.cache
def _make_gather_sum():
  @functools.partial(
      pl.kernel,
      mesh=_mesh(),
      out_type=jax.ShapeDtypeStruct((NCH_A, C, LATENT), jnp.float32),
      scratch_types=[pltpu.VMEM((CPW_A, C), jnp.int32),
                     pltpu.VMEM((CPW_A, C), jnp.int32)]
      + [pltpu.VMEM((C, LATENT), jnp.float32)] * K
      + [pltpu.SemaphoreType.DMA] * (K + 1),
  )
  def gather_sum(tab, ialls, iallr, out, ixs, ixr, *rest):
    bufs = rest[:K]
    sems = rest[K:2 * K]
    wsem = rest[2 * K]
    wid = lax.axis_index("s") * NC + lax.axis_index("c")
    pltpu.sync_copy(ialls.at[wid], ixs)
    pltpu.sync_copy(iallr.at[wid], ixr)

    def rnd(r, carry):
      base = r * K
      hs = [pltpu.async_copy(tab.at[ixs.at[base + b]], bufs[b], sems[b])
            for b in range(K)]
      ha = []
      for b in range(K):
        hs[b].wait()
        ha.append(pltpu.async_copy(tab.at[ixr.at[base + b]], bufs[b],
                                   sems[b], add=True))
      ws = []
      for b in range(K):
        ha[b].wait()
        ws.append(pltpu.async_copy(
            bufs[b], out.at[wid * CPW_A + base + b], wsem))
      for w in ws:
        w.wait()
      return carry

    lax.fori_loop(0, CPW_A // K, rnd, 0)

  return gather_sum


@functools.cache
def _make_scatter_add():
  @functools.partial(
      pl.kernel,
      mesh=_mesh(),
      out_type=jax.ShapeDtypeStruct((NC, NPAD, LATENT), jnp.float32),
      scratch_types=[pltpu.VMEM((CPW_S, CS), jnp.int32)]
      + [pltpu.VMEM((CS, LATENT), jnp.float32)] * KS
      + [pltpu.VMEM_SHARED((NPAD, LATENT), jnp.float32)]
      + [pltpu.SemaphoreType.DMA] * (KS + 1),
  )
  def scatter(el3, ircv, zinit, out, idx_v, *rest):
    bufs = rest[:KS]
    shared = rest[KS]
    sems = rest[KS + 1:2 * KS + 1]
    ssem = rest[2 * KS + 1]
    cid = lax.axis_index("c")
    sid = lax.axis_index("s")
    wid = sid * NC + cid
    pltpu.sync_copy(ircv.at[wid], idx_v)
    pltpu.sync_copy(zinit, shared.at[pl.ds(sid * RPS, RPS)])
    plsc.subcore_barrier()

    def rnd(r, carry):
      base = r * KS
      hs = [pltpu.async_copy(el3.at[wid * CPW_S + base + b], bufs[b], sems[b])
            for b in range(KS)]
      ss = []
      for b in range(KS):
        hs[b].wait()
        ss.append(pltpu.async_copy(
            bufs[b], shared.at[idx_v.at[base + b]], ssem, add=True))
      for s in ss:
        s.wait()
      return carry

    lax.fori_loop(0, CPW_S // KS, rnd, 0)
    plsc.subcore_barrier()
    pltpu.sync_copy(shared.at[pl.ds(sid * RPS, RPS)],
                    out.at[cid, pl.ds(sid * RPS, RPS)])

  return scatter


def _ln(y, g, b):
  m = jnp.mean(y, axis=-1, keepdims=True)
  v = jnp.mean((y - m) ** 2, axis=-1, keepdims=True)
  return (y - m) / jnp.sqrt(v + 1e-5) * g + b


def _dot(a, b):
  return jnp.dot(a, b, preferred_element_type=jnp.float32)


def _rows(bs, cols):
  return pl.BlockSpec((bs, cols), lambda i: (i, 0))


def _rows_off(bs, cols, off):
  return pl.BlockSpec((bs, cols), lambda i: (i + off, 0))


def _rep(shape):
  return pl.BlockSpec(shape, lambda i: (0,) * len(shape))


def _node_enc_body(cp, ty, temb, w1, b1, w2, b2, w3, b3, g, be, ws, wr,
                   nl_o, t_o):
  x = cp[...]
  vel = x[:, 2:12] - x[:, 0:10]
  mr = x[:, 10:12]
  dists = jnp.clip(
      jnp.concatenate([mr - 0.1, 0.9 - mr], axis=1) / R, -1.0, 1.0)
  iot = lax.broadcasted_iota(jnp.int32, (BN, 16), 1).astype(jnp.float32)
  oh = (ty[...] == iot).astype(jnp.float32)
  te = _dot(oh, temb[...])
  feat = jnp.concatenate(
      [vel, dists, te, jnp.zeros((BN, 2), jnp.float32)], axis=1)
  h = jnp.maximum(_dot(feat, w1[...]) + b1[...], 0.0)
  h = jnp.maximum(_dot(h, w2[...]) + b2[...], 0.0)
  nl = _ln(_dot(h, w3[...]) + b3[...], g[...], be[...])
  nl_o[...] = nl
  t_o[0] = _dot(nl, ws[...])
  t_o[1] = _dot(nl, wr[...])


def _edge0_body(gsum, reld, ew1, eb1, ew2, eb2, ew3, eb3, eg, ebe,
                w1e, b1, w2, b2, w3, b3, g, be, el_o):
  rel = reld[:, 0:2] / R
  rd = jnp.sqrt(jnp.sum(rel * rel, axis=1, keepdims=True))
  ef = jnp.concatenate([rel, rd, jnp.zeros((BE, 5), jnp.float32)], axis=1)
  h = jnp.maximum(_dot(ef, ew1[...]) + eb1[...], 0.0)
  h = jnp.maximum(_dot(h, ew2[...]) + eb2[...], 0.0)
  el = _ln(_dot(h, ew3[...]) + eb3[...], eg[...], ebe[...])
  h1 = jnp.maximum(_dot(el, w1e[...]) + gsum[...] + b1[...], 0.0)
  h2 = jnp.maximum(_dot(h1, w2[...]) + b2[...], 0.0)
  eu = _ln(_dot(h2, w3[...]) + b3[...], g[...], be[...])
  el_o[...] = el + eu


def _edge_body(el, gsum, w1e, b1, w2, b2, w3, b3, g, be, el_o):
  elv = el[...]
  h1 = jnp.maximum(_dot(elv, w1e[...]) + gsum[...] + b1[...], 0.0)
  h2 = jnp.maximum(_dot(h1, w2[...]) + b2[...], 0.0)
  eu = _ln(_dot(h2, w3[...]) + b3[...], g[...], be[...])
  el_o[...] = elv + eu


def _node_body(nl, p0, p1, v1n, v1a, b1, v2, b2, v3, b3, g, be, ws, wr,
               nl_o, t_o):
  nlv = nl[...]
  agg = p0[0] + p1[0]
  h1 = jnp.maximum(_dot(nlv, v1n[...]) + _dot(agg, v1a[...]) + b1[...], 0.0)
  h2 = jnp.maximum(_dot(h1, v2[...]) + b2[...], 0.0)
  nu = _ln(_dot(h2, v3[...]) + b3[...], g[...], be[...])
  nl2 = nlv + nu
  nl_o[...] = nl2
  t_o[0] = _dot(nl2, ws[...])
  t_o[1] = _dot(nl2, wr[...])


def _node_last_body(nl, p0, p1, v1n, v1a, b1, v2, b2, v3, b3, g, be, nl_o):
  nlv = nl[...]
  agg = p0[0] + p1[0]
  h1 = jnp.maximum(_dot(nlv, v1n[...]) + _dot(agg, v1a[...]) + b1[...], 0.0)
  h2 = jnp.maximum(_dot(h1, v2[...]) + b2[...], 0.0)
  nu = _ln(_dot(h2, v3[...]) + b3[...], g[...], be[...])
  nl_o[...] = nlv + nu


def _dec_body(cp, nl, d1, db1, d2, db2, d3, db3, out_o):
  x = cp[...]
  h = jnp.maximum(_dot(nl[...], d1[...]) + db1[...], 0.0)
  h = jnp.maximum(_dot(h, d2[...]) + db2[...], 0.0)
  acc = _dot(h, d3[...]) + db3[...]
  mr = x[:, 10:12]
  prev = x[:, 8:10]
  newpos = mr + (mr - prev) + acc[:, 0:2]
  out_o[...] = jnp.concatenate(
      [newpos, jnp.zeros((BN, 6), jnp.float32)], axis=1)


_W128 = _rep((LATENT, LATENT))
_B128 = _rep((1, LATENT))
_F32 = jax.ShapeDtypeStruct

_TSPEC = pl.BlockSpec((2, BN, LATENT), lambda i: (0, i, 0))
_TSHAPE = _F32((2, N, LATENT), jnp.float32)


def _node_enc_call(cp, ty, temb, args):
  return pl.pallas_call(
      _node_enc_body,
      grid=(N // BN,),
      in_specs=[_rows(BN, 12), _rows(BN, 1), _rep((16, 16)),
                _rep((32, LATENT)), _B128, _W128, _B128, _W128, _B128,
                _B128, _B128, _W128, _W128],
      out_specs=[_rows(BN, LATENT), _TSPEC],
      out_shape=[_F32((N, LATENT), jnp.float32), _TSHAPE],
  )(cp, ty, temb, *args)


def _edge0_call(gsum, reld, enc_args, st_args):
  return pl.pallas_call(
      _edge0_body,
      grid=(EB,),
      in_specs=[_rows(BE, LATENT), _rows(BE, LATENT),
                _rep((8, LATENT)), _B128, _W128, _B128, _W128, _B128,
                _B128, _B128,
                _W128, _B128, _W128, _B128, _W128, _B128, _B128, _B128],
      out_specs=_rows(BE, LATENT),
      out_shape=_F32((E, LATENT), jnp.float32),
  )(gsum, reld, *enc_args, *st_args)


def _edge_call(el, gsum, st_args):
  return pl.pallas_call(
      _edge_body,
      grid=(EB,),
      in_specs=[_rows(BE, LATENT), _rows(BE, LATENT),
                _W128, _B128, _W128, _B128, _W128, _B128, _B128, _B128],
      out_specs=_rows(BE, LATENT),
      out_shape=_F32((E, LATENT), jnp.float32),
  )(el, gsum, *st_args)


def _node_call(nl, p0, p1, st_args, ws, wr):
  return pl.pallas_call(
      _node_body,
      grid=(N // BN,),
      in_specs=[_rows(BN, LATENT),
                pl.BlockSpec((1, BN, LATENT), lambda i: (0, i, 0)),
                pl.BlockSpec((1, BN, LATENT), lambda i: (1, i, 0)),
                _W128, _W128, _B128, _W128, _B128, _W128, _B128, _B128,
                _B128, _W128, _W128],
      out_specs=[_rows(BN, LATENT), _TSPEC],
      out_shape=[_F32((N, LATENT), jnp.float32), _TSHAPE],
  )(nl, p0, p1, *st_args, ws, wr)


def _node_last_call(nl, p0, p1, st_args):
  return pl.pallas_call(
      _node_last_body,
      grid=(N // BN,),
      in_specs=[_rows(BN, LATENT),
                pl.BlockSpec((1, BN, LATENT), lambda i: (0, i, 0)),
                pl.BlockSpec((1, BN, LATENT), lambda i: (1, i, 0)),
                _W128, _W128, _B128, _W128, _B128, _W128, _B128, _B128,
                _B128],
      out_specs=_rows(BN, LATENT),
      out_shape=_F32((N, LATENT), jnp.float32),
  )(nl, p0, p1, *st_args)


def _dec_call(cp, nl, args):
  return pl.pallas_call(
      _dec_body,
      grid=(N // BN,),
      in_specs=[_rows(BN, 12), _rows(BN, LATENT),
                _W128, _B128, _W128, _B128, _rep((LATENT, 8)), _rep((1, 8))],
      out_specs=_rows(BN, 8),
      out_shape=_F32((N, 8), jnp.float32),
  )(cp, nl, *args)


def _b(x):
  return x.reshape(1, -1)


def kernel(current_positions, particle_types, edge_index, params):
  cp = current_positions.reshape(N, SEQ * DIM)
  ty = particle_types.astype(jnp.float32).reshape(N, 1)
  snd = edge_index[0].astype(jnp.int32)
  rcv = edge_index[1].astype(jnp.int32)
  ialls = snd.reshape(NW, CPW_A, C)
  iallr = (rcv + N).reshape(NW, CPW_A, C)
  ircv3 = rcv.reshape(NW, CPW_S, CS)
  zinit = jnp.zeros((RPS, LATENT), jnp.float32)

  temb = jnp.pad(params["type_emb"], ((0, 16 - TYPES), (0, 0)))
  ne = params["node_enc"]
  ne_w1 = jnp.pad(ne["mlp"][0]["W"], ((0, 2), (0, 0)))
  ne_args = (ne_w1, _b(ne["mlp"][0]["b"]), ne["mlp"][1]["W"],
             _b(ne["mlp"][1]["b"]), ne["mlp"][2]["W"], _b(ne["mlp"][2]["b"]),
             _b(ne["g"]), _b(ne["be"]))
  ee = params["edge_enc"]
  ee_w1 = jnp.pad(ee["mlp"][0]["W"], ((0, 5), (0, 0)))
  ee_args = (ee_w1, _b(ee["mlp"][0]["b"]), ee["mlp"][1]["W"],
             _b(ee["mlp"][1]["b"]), ee["mlp"][2]["W"], _b(ee["mlp"][2]["b"]),
             _b(ee["g"]), _b(ee["be"]))

  esplit, eargs, nargs = [], [], []
  for st in params["proc"]:
    ew = st["edge"]["mlp"][0]["W"]
    esplit.append((ew[:LATENT], ew[LATENT:2 * LATENT], ew[2 * LATENT:]))
    em = st["edge"]
    eargs.append((em["mlp"][0]["W"][:LATENT], _b(em["mlp"][0]["b"]),
                  em["mlp"][1]["W"], _b(em["mlp"][1]["b"]),
                  em["mlp"][2]["W"], _b(em["mlp"][2]["b"]),
                  _b(em["g"]), _b(em["be"])))
    nm = st["node"]
    nw = nm["mlp"][0]["W"]
    nargs.append((nw[:LATENT], nw[LATENT:], _b(nm["mlp"][0]["b"]),
                  nm["mlp"][1]["W"], _b(nm["mlp"][1]["b"]),
                  nm["mlp"][2]["W"], _b(nm["mlp"][2]["b"]),
                  _b(nm["g"]), _b(nm["be"])))

  dm = params["dec"]["mlp"]
  d3 = jnp.pad(dm[2]["W"], ((0, 0), (0, 8 - DIM)))
  db3 = _b(jnp.pad(dm[2]["b"], (0, 8 - DIM)))
  dec_args = (dm[0]["W"], _b(dm[0]["b"]), dm[1]["W"], _b(dm[1]["b"]),
              d3, db3)

  nl, tstack = _node_enc_call(
      cp, ty, temb, ne_args + (esplit[0][1], esplit[0][2]))
  pos_tab = jnp.pad(cp[:, 10:12], ((0, 0), (0, LATENT - DIM)))
  posneg = jnp.concatenate([pos_tab, -pos_tab], axis=0)
  gsum = _make_gather_sum()(tstack.reshape(2 * N, LATENT), ialls, iallr)
  reld = _make_gather_sum()(posneg, ialls, iallr)
  el = _edge0_call(gsum.reshape(E, LATENT), reld.reshape(E, LATENT),
                   ee_args, eargs[0])

  for t in range(STEPS):
    if t > 0:
      gsum = _make_gather_sum()(tstack.reshape(2 * N, LATENT), ialls, iallr)
      el = _edge_call(el, gsum.reshape(E, LATENT), eargs[t])
    parts = _make_scatter_add()(el.reshape(NCH_S, CS, LATENT), ircv3, zinit)
    if t < STEPS - 1:
      nl, tstack = _node_call(nl, parts, parts, nargs[t],
                              esplit[t + 1][1], esplit[t + 1][2])
    else:
      nl = _node_last_call(nl, parts, parts, nargs[t])

  out = _dec_call(cp, nl, dec_args)
  return out[:, :DIM]


# speedup vs baseline: 1.0762x; 1.0539x over previous
import functools

import jax
import jax.numpy as jnp
from jax import lax
from jax.experimental import pallas as pl
from jax.experimental.pallas import tpu as pltpu
from jax.experimental.pallas import tpu_sc as plsc

N = 10000
E = 160000
SEQ = 6
DIM = 2
LATENT = 128
STEPS = 10
TYPES = 9
R = 0.015

NC = 2
NS = 16
NW = NC * NS
C = 100
K = 5
KS = 5
CS = 50

NCH_A = E // C
CPW_A = NCH_A // NW
NCH_S = E // CS
CPW_S = NCH_S // NW

NPAD = 10240
RPS = NPAD // NS

BN = 2000
BE = 4000
EB = E // BE


@functools.cache
def _mesh():
  return plsc.VectorSubcoreMesh(core_axis_name="c", subcore_axis_name="s",
                                num_cores=NC, num_subcores=NS)


@functools.cache
def _make_gather_sum():
  @functools.partial(
      pl.kernel,
      mesh=_mesh(),
      out_type=jax.ShapeDtypeStruct((NCH_A, C, LATENT), jnp.float32),
      scratch_types=[pltpu.VMEM((CPW_A, C), jnp.int32),
                     pltpu.VMEM((CPW_A, C), jnp.int32)]
      + [pltpu.VMEM((C, LATENT), jnp.float32)] * K
      + [pltpu.SemaphoreType.DMA] * (K + 1),
  )
  def gather_sum(tab, ialls, iallr, out, ixs, ixr, *rest):
    bufs = rest[:K]
    sems = rest[K:2 * K]
    wsem = rest[2 * K]
    wid = lax.axis_index("s") * NC + lax.axis_index("c")
    pltpu.sync_copy(ialls.at[wid], ixs)
    pltpu.sync_copy(iallr.at[wid], ixr)

    def rnd(r, carry):
      base = r * K
      hs = [pltpu.async_copy(tab.at[ixs.at[base + b]], bufs[b], sems[b])
            for b in range(K)]
      ha = []
      for b in range(K):
        hs[b].wait()
        ha.append(pltpu.async_copy(tab.at[ixr.at[base + b]], bufs[b],
                                   sems[b], add=True))
      ws = []
      for b in range(K):
        ha[b].wait()
        ws.append(pltpu.async_copy(
            bufs[b], out.at[wid * CPW_A + base + b], wsem))
      for w in ws:
        w.wait()
      return carry

    lax.fori_loop(0, CPW_A // K, rnd, 0)

  return gather_sum


@functools.cache
def _make_scatter_add():
  @functools.partial(
      pl.kernel,
      mesh=_mesh(),
      out_type=jax.ShapeDtypeStruct((NC, NPAD, LATENT), jnp.float32),
      scratch_types=[pltpu.VMEM((CPW_S, CS), jnp.int32)]
      + [pltpu.VMEM((CS, LATENT), jnp.float32)] * KS
      + [pltpu.VMEM_SHARED((NPAD, LATENT), jnp.float32)]
      + [pltpu.SemaphoreType.DMA] * (KS + 1),
  )
  def scatter(el3, ircv, zinit, out, idx_v, *rest):
    bufs = rest[:KS]
    shared = rest[KS]
    sems = rest[KS + 1:2 * KS + 1]
    ssem = rest[2 * KS + 1]
    cid = lax.axis_index("c")
    sid = lax.axis_index("s")
    wid = sid * NC + cid
    pltpu.sync_copy(ircv.at[wid], idx_v)
    pltpu.sync_copy(zinit, shared.at[pl.ds(sid * RPS, RPS)])
    plsc.subcore_barrier()

    def rnd(r, carry):
      base = r * KS
      hs = [pltpu.async_copy(el3.at[wid * CPW_S + base + b], bufs[b], sems[b])
            for b in range(KS)]
      ss = []
      for b in range(KS):
        hs[b].wait()
        ss.append(pltpu.async_copy(
            bufs[b], shared.at[idx_v.at[base + b]], ssem, add=True))
      for s in ss:
        s.wait()
      return carry

    lax.fori_loop(0, CPW_S // KS, rnd, 0)
    plsc.subcore_barrier()
    pltpu.sync_copy(shared.at[pl.ds(sid * RPS, RPS)],
                    out.at[cid, pl.ds(sid * RPS, RPS)])

  return scatter


def _ln(y, g, b):
  m = jnp.mean(y, axis=-1, keepdims=True)
  v = jnp.mean((y - m) ** 2, axis=-1, keepdims=True)
  return (y - m) / jnp.sqrt(v + 1e-5) * g + b


def _dot(a, b):
  return jnp.dot(a, b, preferred_element_type=jnp.float32)


def _rows(bs, cols):
  return pl.BlockSpec((bs, cols), lambda i: (i, 0))


def _rows_off(bs, cols, off):
  return pl.BlockSpec((bs, cols), lambda i: (i + off, 0))


def _rep(shape):
  return pl.BlockSpec(shape, lambda i: (0,) * len(shape))


def _node_enc_body(cp, ty, temb, w1, b1, w2, b2, w3, b3, g, be, ws, wr,
                   nl_o, t_o):
  x = cp[...]
  vel = x[:, 2:12] - x[:, 0:10]
  mr = x[:, 10:12]
  dists = jnp.clip(
      jnp.concatenate([mr - 0.1, 0.9 - mr], axis=1) / R, -1.0, 1.0)
  iot = lax.broadcasted_iota(jnp.int32, (BN, 16), 1).astype(jnp.float32)
  oh = (ty[...] == iot).astype(jnp.float32)
  te = _dot(oh, temb[...])
  feat = jnp.concatenate(
      [vel, dists, te, jnp.zeros((BN, 2), jnp.float32)], axis=1)
  h = jnp.maximum(_dot(feat, w1[...]) + b1[...], 0.0)
  h = jnp.maximum(_dot(h, w2[...]) + b2[...], 0.0)
  nl = _ln(_dot(h, w3[...]) + b3[...], g[...], be[...])
  nl_o[...] = nl
  t_o[0] = _dot(nl, ws[...])
  t_o[1] = _dot(nl, wr[...])


def _edge0_body(gsum, reld, ew1, eb1, ew2, eb2, ew3, eb3, eg, ebe,
                w1e, b1, w2, b2, w3, b3, g, be, el_o):
  rel = reld[:, 0:2] / R
  rd = jnp.sqrt(jnp.sum(rel * rel, axis=1, keepdims=True))
  ef = jnp.concatenate([rel, rd, jnp.zeros((BE, 5), jnp.float32)], axis=1)
  h = jnp.maximum(_dot(ef, ew1[...]) + eb1[...], 0.0)
  h = jnp.maximum(_dot(h, ew2[...]) + eb2[...], 0.0)
  el = _ln(_dot(h, ew3[...]) + eb3[...], eg[...], ebe[...])
  h1 = jnp.maximum(_dot(el, w1e[...]) + gsum[...] + b1[...], 0.0)
  h2 = jnp.maximum(_dot(h1, w2[...]) + b2[...], 0.0)
  eu = _ln(_dot(h2, w3[...]) + b3[...], g[...], be[...])
  el_o[...] = el + eu


def _edge_body(el, gsum, w1e, b1, w2, b2, w3, b3, g, be, el_o):
  elv = el[...]
  h1 = jnp.maximum(_dot(elv, w1e[...]) + gsum[...] + b1[...], 0.0)
  h2 = jnp.maximum(_dot(h1, w2[...]) + b2[...], 0.0)
  eu = _ln(_dot(h2, w3[...]) + b3[...], g[...], be[...])
  el_o[...] = elv + eu


def _node_body(nl, p0, p1, v1n, v1a, b1, v2, b2, v3, b3, g, be, ws, wr,
               nl_o, t_o):
  nlv = nl[...]
  agg = p0[0] + p1[0]
  h1 = jnp.maximum(_dot(nlv, v1n[...]) + _dot(agg, v1a[...]) + b1[...], 0.0)
  h2 = jnp.maximum(_dot(h1, v2[...]) + b2[...], 0.0)
  nu = _ln(_dot(h2, v3[...]) + b3[...], g[...], be[...])
  nl2 = nlv + nu
  nl_o[...] = nl2
  t_o[0] = _dot(nl2, ws[...])
  t_o[1] = _dot(nl2, wr[...])


def _node_last_body(nl, p0, p1, cp, v1n, v1a, b1, v2, b2, v3, b3, g, be,
                    d1, db1, d2, db2, d3, db3, out_o):
  nlv = nl[...]
  agg = p0[0] + p1[0]
  h1 = jnp.maximum(_dot(nlv, v1n[...]) + _dot(agg, v1a[...]) + b1[...], 0.0)
  h2 = jnp.maximum(_dot(h1, v2[...]) + b2[...], 0.0)
  nu = _ln(_dot(h2, v3[...]) + b3[...], g[...], be[...])
  nl2 = nlv + nu
  h = jnp.maximum(_dot(nl2, d1[...]) + db1[...], 0.0)
  h = jnp.maximum(_dot(h, d2[...]) + db2[...], 0.0)
  acc = _dot(h, d3[...]) + db3[...]
  x = cp[...]
  mr = x[:, 10:12]
  prev = x[:, 8:10]
  newpos = mr + (mr - prev) + acc[:, 0:2]
  out_o[...] = jnp.concatenate(
      [newpos, jnp.zeros((BN, 6), jnp.float32)], axis=1)


_W128 = _rep((LATENT, LATENT))
_B128 = _rep((1, LATENT))
_F32 = jax.ShapeDtypeStruct

_TSPEC = pl.BlockSpec((2, BN, LATENT), lambda i: (0, i, 0))
_TSHAPE = _F32((2, N, LATENT), jnp.float32)


def _node_enc_call(cp, ty, temb, args):
  return pl.pallas_call(
      _node_enc_body,
      grid=(N // BN,),
      in_specs=[_rows(BN, 12), _rows(BN, 1), _rep((16, 16)),
                _rep((32, LATENT)), _B128, _W128, _B128, _W128, _B128,
                _B128, _B128, _W128, _W128],
      out_specs=[_rows(BN, LATENT), _TSPEC],
      out_shape=[_F32((N, LATENT), jnp.float32), _TSHAPE],
  )(cp, ty, temb, *args)


def _edge0_call(gsum, reld, enc_args, st_args):
  return pl.pallas_call(
      _edge0_body,
      grid=(EB,),
      in_specs=[_rows(BE, LATENT), _rows(BE, LATENT),
                _rep((8, LATENT)), _B128, _W128, _B128, _W128, _B128,
                _B128, _B128,
                _W128, _B128, _W128, _B128, _W128, _B128, _B128, _B128],
      out_specs=_rows(BE, LATENT),
      out_shape=_F32((E, LATENT), jnp.float32),
  )(gsum, reld, *enc_args, *st_args)


def _edge_call(el, gsum, st_args):
  return pl.pallas_call(
      _edge_body,
      grid=(EB,),
      in_specs=[_rows(BE, LATENT), _rows(BE, LATENT),
                _W128, _B128, _W128, _B128, _W128, _B128, _B128, _B128],
      out_specs=_rows(BE, LATENT),
      out_shape=_F32((E, LATENT), jnp.float32),
  )(el, gsum, *st_args)


def _node_call(nl, p0, p1, st_args, ws, wr):
  return pl.pallas_call(
      _node_body,
      grid=(N // BN,),
      in_specs=[_rows(BN, LATENT),
                pl.BlockSpec((1, BN, LATENT), lambda i: (0, i, 0)),
                pl.BlockSpec((1, BN, LATENT), lambda i: (1, i, 0)),
                _W128, _W128, _B128, _W128, _B128, _W128, _B128, _B128,
                _B128, _W128, _W128],
      out_specs=[_rows(BN, LATENT), _TSPEC],
      out_shape=[_F32((N, LATENT), jnp.float32), _TSHAPE],
  )(nl, p0, p1, *st_args, ws, wr)


def _node_last_call(nl, p0, p1, cp, st_args, dec_args):
  return pl.pallas_call(
      _node_last_body,
      grid=(N // BN,),
      in_specs=[_rows(BN, LATENT),
                pl.BlockSpec((1, BN, LATENT), lambda i: (0, i, 0)),
                pl.BlockSpec((1, BN, LATENT), lambda i: (1, i, 0)),
                _rows(BN, 12),
                _W128, _W128, _B128, _W128, _B128, _W128, _B128, _B128,
                _B128,
                _W128, _B128, _W128, _B128, _rep((LATENT, 8)), _rep((1, 8))],
      out_specs=_rows(BN, 8),
      out_shape=_F32((N, 8), jnp.float32),
  )(nl, p0, p1, cp, *st_args, *dec_args)


def _b(x):
  return x.reshape(1, -1)


def kernel(current_positions, particle_types, edge_index, params):
  cp = current_positions.reshape(N, SEQ * DIM)
  ty = particle_types.astype(jnp.float32).reshape(N, 1)
  snd = edge_index[0].astype(jnp.int32)
  rcv = edge_index[1].astype(jnp.int32)
  ialls = snd.reshape(NW, CPW_A, C)
  iallr = (rcv + N).reshape(NW, CPW_A, C)
  ircv3 = rcv.reshape(NW, CPW_S, CS)
  zinit = jnp.zeros((RPS, LATENT), jnp.float32)

  temb = jnp.pad(params["type_emb"], ((0, 16 - TYPES), (0, 0)))
  ne = params["node_enc"]
  ne_w1 = jnp.pad(ne["mlp"][0]["W"], ((0, 2), (0, 0)))
  ne_args = (ne_w1, _b(ne["mlp"][0]["b"]), ne["mlp"][1]["W"],
             _b(ne["mlp"][1]["b"]), ne["mlp"][2]["W"], _b(ne["mlp"][2]["b"]),
             _b(ne["g"]), _b(ne["be"]))
  ee = params["edge_enc"]
  ee_w1 = jnp.pad(ee["mlp"][0]["W"], ((0, 5), (0, 0)))
  ee_args = (ee_w1, _b(ee["mlp"][0]["b"]), ee["mlp"][1]["W"],
             _b(ee["mlp"][1]["b"]), ee["mlp"][2]["W"], _b(ee["mlp"][2]["b"]),
             _b(ee["g"]), _b(ee["be"]))

  esplit, eargs, nargs = [], [], []
  for st in params["proc"]:
    ew = st["edge"]["mlp"][0]["W"]
    esplit.append((ew[:LATENT], ew[LATENT:2 * LATENT], ew[2 * LATENT:]))
    em = st["edge"]
    eargs.append((em["mlp"][0]["W"][:LATENT], _b(em["mlp"][0]["b"]),
                  em["mlp"][1]["W"], _b(em["mlp"][1]["b"]),
                  em["mlp"][2]["W"], _b(em["mlp"][2]["b"]),
                  _b(em["g"]), _b(em["be"])))
    nm = st["node"]
    nw = nm["mlp"][0]["W"]
    nargs.append((nw[:LATENT], nw[LATENT:], _b(nm["mlp"][0]["b"]),
                  nm["mlp"][1]["W"], _b(nm["mlp"][1]["b"]),
                  nm["mlp"][2]["W"], _b(nm["mlp"][2]["b"]),
                  _b(nm["g"]), _b(nm["be"])))

  dm = params["dec"]["mlp"]
  d3 = jnp.pad(dm[2]["W"], ((0, 0), (0, 8 - DIM)))
  db3 = _b(jnp.pad(dm[2]["b"], (0, 8 - DIM)))
  dec_args = (dm[0]["W"], _b(dm[0]["b"]), dm[1]["W"], _b(dm[1]["b"]),
              d3, db3)

  nl, tstack = _node_enc_call(
      cp, ty, temb, ne_args + (esplit[0][1], esplit[0][2]))
  pos_tab = jnp.pad(cp[:, 10:12], ((0, 0), (0, LATENT - DIM)))
  posneg = jnp.concatenate([pos_tab, -pos_tab], axis=0)
  gsum = _make_gather_sum()(tstack.reshape(2 * N, LATENT), ialls, iallr)
  reld = _make_gather_sum()(posneg, ialls, iallr)
  el = _edge0_call(gsum.reshape(E, LATENT), reld.reshape(E, LATENT),
                   ee_args, eargs[0])

  for t in range(STEPS):
    if t > 0:
      gsum = _make_gather_sum()(tstack.reshape(2 * N, LATENT), ialls, iallr)
      el = _edge_call(el, gsum.reshape(E, LATENT), eargs[t])
    parts = _make_scatter_add()(el.reshape(NCH_S, CS, LATENT), ircv3, zinit)
    if t < STEPS - 1:
      nl, tstack = _node_call(nl, parts, parts, nargs[t],
                              esplit[t + 1][1], esplit[t + 1][2])
    else:
      out = _node_last_call(nl, parts, parts, cp, nargs[t], dec_args)

  return out[:, :DIM]


# speedup vs baseline: 1.0980x; 1.0203x over previous
import functools

import jax
import jax.numpy as jnp
from jax import lax
from jax.experimental import pallas as pl
from jax.experimental.pallas import tpu as pltpu
from jax.experimental.pallas import tpu_sc as plsc

N = 10000
E = 160000
SEQ = 6
DIM = 2
LATENT = 128
STEPS = 10
TYPES = 9
R = 0.015

NC = 2
NS = 16
NW = NC * NS
C = 100
K = 5
KS = 5
CS = 50

NCH_A = E // C
CPW_A = NCH_A // NW
NCH_S = E // CS
CPW_S = NCH_S // NW

NPAD = 10240
RPS = NPAD // NS

BN = 2000
BE = 8000
EB = E // BE


@functools.cache
def _mesh():
  return plsc.VectorSubcoreMesh(core_axis_name="c", subcore_axis_name="s",
                                num_cores=NC, num_subcores=NS)


@functools.cache
def _make_gather_sum():
  @functools.partial(
      pl.kernel,
      mesh=_mesh(),
      out_type=jax.ShapeDtypeStruct((NCH_A, C, LATENT), jnp.float32),
      scratch_types=[pltpu.VMEM((CPW_A, C), jnp.int32),
                     pltpu.VMEM((CPW_A, C), jnp.int32)]
      + [pltpu.VMEM((C, LATENT), jnp.float32)] * K
      + [pltpu.SemaphoreType.DMA] * (K + 1),
  )
  def gather_sum(tab, ialls, iallr, out, ixs, ixr, *rest):
    bufs = rest[:K]
    sems = rest[K:2 * K]
    wsem = rest[2 * K]
    wid = lax.axis_index("s") * NC + lax.axis_index("c")
    pltpu.sync_copy(ialls.at[wid], ixs)
    pltpu.sync_copy(iallr.at[wid], ixr)

    def rnd(r, carry):
      base = r * K
      hs = [pltpu.async_copy(tab.at[ixs.at[base + b]], bufs[b], sems[b])
            for b in range(K)]
      ha = []
      for b in range(K):
        hs[b].wait()
        ha.append(pltpu.async_copy(tab.at[ixr.at[base + b]], bufs[b],
                                   sems[b], add=True))
      ws = []
      for b in range(K):
        ha[b].wait()
        ws.append(pltpu.async_copy(
            bufs[b], out.at[wid * CPW_A + base + b], wsem))
      for w in ws:
        w.wait()
      return carry

    lax.fori_loop(0, CPW_A // K, rnd, 0)

  return gather_sum


@functools.cache
def _make_scatter_add():
  @functools.partial(
      pl.kernel,
      mesh=_mesh(),
      out_type=jax.ShapeDtypeStruct((NC, NPAD, LATENT), jnp.float32),
      scratch_types=[pltpu.VMEM((CPW_S, CS), jnp.int32)]
      + [pltpu.VMEM((CS, LATENT), jnp.float32)] * KS
      + [pltpu.VMEM_SHARED((NPAD, LATENT), jnp.float32)]
      + [pltpu.SemaphoreType.DMA] * (KS + 1),
  )
  def scatter(el3, ircv, zinit, out, idx_v, *rest):
    bufs = rest[:KS]
    shared = rest[KS]
    sems = rest[KS + 1:2 * KS + 1]
    ssem = rest[2 * KS + 1]
    cid = lax.axis_index("c")
    sid = lax.axis_index("s")
    wid = sid * NC + cid
    pltpu.sync_copy(ircv.at[wid], idx_v)
    pltpu.sync_copy(zinit, shared.at[pl.ds(sid * RPS, RPS)])
    plsc.subcore_barrier()

    def rnd(r, carry):
      base = r * KS
      hs = [pltpu.async_copy(el3.at[wid * CPW_S + base + b], bufs[b], sems[b])
            for b in range(KS)]
      ss = []
      for b in range(KS):
        hs[b].wait()
        ss.append(pltpu.async_copy(
            bufs[b], shared.at[idx_v.at[base + b]], ssem, add=True))
      for s in ss:
        s.wait()
      return carry

    lax.fori_loop(0, CPW_S // KS, rnd, 0)
    plsc.subcore_barrier()
    pltpu.sync_copy(shared.at[pl.ds(sid * RPS, RPS)],
                    out.at[cid, pl.ds(sid * RPS, RPS)])

  return scatter


def _ln(y, g, b):
  m = jnp.mean(y, axis=-1, keepdims=True)
  v = jnp.mean((y - m) ** 2, axis=-1, keepdims=True)
  return (y - m) / jnp.sqrt(v + 1e-5) * g + b


def _dot(a, b):
  return jnp.dot(a, b, preferred_element_type=jnp.float32)


def _rows(bs, cols):
  return pl.BlockSpec((bs, cols), lambda i: (i, 0))


def _rows_off(bs, cols, off):
  return pl.BlockSpec((bs, cols), lambda i: (i + off, 0))


def _rep(shape):
  return pl.BlockSpec(shape, lambda i: (0,) * len(shape))


def _node_enc_body(cp, ty, temb, w1, b1, w2, b2, w3, b3, g, be, ws, wr,
                   nl_o, t_o):
  x = cp[...]
  vel = x[:, 2:12] - x[:, 0:10]
  mr = x[:, 10:12]
  dists = jnp.clip(
      jnp.concatenate([mr - 0.1, 0.9 - mr], axis=1) / R, -1.0, 1.0)
  iot = lax.broadcasted_iota(jnp.int32, (BN, 16), 1).astype(jnp.float32)
  oh = (ty[...] == iot).astype(jnp.float32)
  te = _dot(oh, temb[...])
  feat = jnp.concatenate(
      [vel, dists, te, jnp.zeros((BN, 2), jnp.float32)], axis=1)
  h = jnp.maximum(_dot(feat, w1[...]) + b1[...], 0.0)
  h = jnp.maximum(_dot(h, w2[...]) + b2[...], 0.0)
  nl = _ln(_dot(h, w3[...]) + b3[...], g[...], be[...])
  nl_o[...] = nl
  t_o[0] = _dot(nl, ws[...])
  t_o[1] = _dot(nl, wr[...])


def _edge0_body(gsum, reld, ew1, eb1, ew2, eb2, ew3, eb3, eg, ebe,
                w1e, b1, w2, b2, w3, b3, g, be, el_o):
  rel = reld[:, 0:2] / R
  rd = jnp.sqrt(jnp.sum(rel * rel, axis=1, keepdims=True))
  ef = jnp.concatenate([rel, rd, jnp.zeros((BE, 5), jnp.float32)], axis=1)
  h = jnp.maximum(_dot(ef, ew1[...]) + eb1[...], 0.0)
  h = jnp.maximum(_dot(h, ew2[...]) + eb2[...], 0.0)
  el = _ln(_dot(h, ew3[...]) + eb3[...], eg[...], ebe[...])
  h1 = jnp.maximum(_dot(el, w1e[...]) + gsum[...] + b1[...], 0.0)
  h2 = jnp.maximum(_dot(h1, w2[...]) + b2[...], 0.0)
  eu = _ln(_dot(h2, w3[...]) + b3[...], g[...], be[...])
  el_o[...] = el + eu


def _edge_body(el, gsum, w1e, b1, w2, b2, w3, b3, g, be, el_o):
  elv = el[...]
  h1 = jnp.maximum(_dot(elv, w1e[...]) + gsum[...] + b1[...], 0.0)
  h2 = jnp.maximum(_dot(h1, w2[...]) + b2[...], 0.0)
  eu = _ln(_dot(h2, w3[...]) + b3[...], g[...], be[...])
  el_o[...] = elv + eu


def _node_body(nl, p0, p1, v1n, v1a, b1, v2, b2, v3, b3, g, be, ws, wr,
               nl_o, t_o):
  nlv = nl[...]
  agg = p0[0] + p1[0]
  h1 = jnp.maximum(_dot(nlv, v1n[...]) + _dot(agg, v1a[...]) + b1[...], 0.0)
  h2 = jnp.maximum(_dot(h1, v2[...]) + b2[...], 0.0)
  nu = _ln(_dot(h2, v3[...]) + b3[...], g[...], be[...])
  nl2 = nlv + nu
  nl_o[...] = nl2
  t_o[0] = _dot(nl2, ws[...])
  t_o[1] = _dot(nl2, wr[...])


def _node_last_body(nl, p0, p1, cp, v1n, v1a, b1, v2, b2, v3, b3, g, be,
                    d1, db1, d2, db2, d3, db3, out_o):
  nlv = nl[...]
  agg = p0[0] + p1[0]
  h1 = jnp.maximum(_dot(nlv, v1n[...]) + _dot(agg, v1a[...]) + b1[...], 0.0)
  h2 = jnp.maximum(_dot(h1, v2[...]) + b2[...], 0.0)
  nu = _ln(_dot(h2, v3[...]) + b3[...], g[...], be[...])
  nl2 = nlv + nu
  h = jnp.maximum(_dot(nl2, d1[...]) + db1[...], 0.0)
  h = jnp.maximum(_dot(h, d2[...]) + db2[...], 0.0)
  acc = _dot(h, d3[...]) + db3[...]
  x = cp[...]
  mr = x[:, 10:12]
  prev = x[:, 8:10]
  newpos = mr + (mr - prev) + acc[:, 0:2]
  out_o[...] = jnp.concatenate(
      [newpos, jnp.zeros((BN, 6), jnp.float32)], axis=1)


_W128 = _rep((LATENT, LATENT))
_B128 = _rep((1, LATENT))
_F32 = jax.ShapeDtypeStruct

_TSPEC = pl.BlockSpec((2, BN, LATENT), lambda i: (0, i, 0))
_TSHAPE = _F32((2, N, LATENT), jnp.float32)


def _node_enc_call(cp, ty, temb, args):
  return pl.pallas_call(
      _node_enc_body,
      grid=(N // BN,),
      in_specs=[_rows(BN, 12), _rows(BN, 1), _rep((16, 16)),
                _rep((32, LATENT)), _B128, _W128, _B128, _W128, _B128,
                _B128, _B128, _W128, _W128],
      out_specs=[_rows(BN, LATENT), _TSPEC],
      out_shape=[_F32((N, LATENT), jnp.float32), _TSHAPE],
  )(cp, ty, temb, *args)


def _edge0_call(gsum, reld, enc_args, st_args):
  return pl.pallas_call(
      _edge0_body,
      grid=(EB,),
      in_specs=[_rows(BE, LATENT), _rows(BE, LATENT),
                _rep((8, LATENT)), _B128, _W128, _B128, _W128, _B128,
                _B128, _B128,
                _W128, _B128, _W128, _B128, _W128, _B128, _B128, _B128],
      out_specs=_rows(BE, LATENT),
      out_shape=_F32((E, LATENT), jnp.float32),
  )(gsum, reld, *enc_args, *st_args)


def _edge_call(el, gsum, st_args):
  return pl.pallas_call(
      _edge_body,
      grid=(EB,),
      in_specs=[_rows(BE, LATENT), _rows(BE, LATENT),
                _W128, _B128, _W128, _B128, _W128, _B128, _B128, _B128],
      out_specs=_rows(BE, LATENT),
      out_shape=_F32((E, LATENT), jnp.float32),
  )(el, gsum, *st_args)


def _node_call(nl, p0, p1, st_args, ws, wr):
  return pl.pallas_call(
      _node_body,
      grid=(N // BN,),
      in_specs=[_rows(BN, LATENT),
                pl.BlockSpec((1, BN, LATENT), lambda i: (0, i, 0)),
                pl.BlockSpec((1, BN, LATENT), lambda i: (1, i, 0)),
                _W128, _W128, _B128, _W128, _B128, _W128, _B128, _B128,
                _B128, _W128, _W128],
      out_specs=[_rows(BN, LATENT), _TSPEC],
      out_shape=[_F32((N, LATENT), jnp.float32), _TSHAPE],
  )(nl, p0, p1, *st_args, ws, wr)


def _node_last_call(nl, p0, p1, cp, st_args, dec_args):
  return pl.pallas_call(
      _node_last_body,
      grid=(N // BN,),
      in_specs=[_rows(BN, LATENT),
                pl.BlockSpec((1, BN, LATENT), lambda i: (0, i, 0)),
                pl.BlockSpec((1, BN, LATENT), lambda i: (1, i, 0)),
                _rows(BN, 12),
                _W128, _W128, _B128, _W128, _B128, _W128, _B128, _B128,
                _B128,
                _W128, _B128, _W128, _B128, _rep((LATENT, 8)), _rep((1, 8))],
      out_specs=_rows(BN, 8),
      out_shape=_F32((N, 8), jnp.float32),
  )(nl, p0, p1, cp, *st_args, *dec_args)


def _b(x):
  return x.reshape(1, -1)


def kernel(current_positions, particle_types, edge_index, params):
  cp = current_positions.reshape(N, SEQ * DIM)
  ty = particle_types.astype(jnp.float32).reshape(N, 1)
  snd = edge_index[0].astype(jnp.int32)
  rcv = edge_index[1].astype(jnp.int32)
  ialls = snd.reshape(NW, CPW_A, C)
  iallr = (rcv + N).reshape(NW, CPW_A, C)
  ircv3 = rcv.reshape(NW, CPW_S, CS)
  zinit = jnp.zeros((RPS, LATENT), jnp.float32)

  temb = jnp.pad(params["type_emb"], ((0, 16 - TYPES), (0, 0)))
  ne = params["node_enc"]
  ne_w1 = jnp.pad(ne["mlp"][0]["W"], ((0, 2), (0, 0)))
  ne_args = (ne_w1, _b(ne["mlp"][0]["b"]), ne["mlp"][1]["W"],
             _b(ne["mlp"][1]["b"]), ne["mlp"][2]["W"], _b(ne["mlp"][2]["b"]),
             _b(ne["g"]), _b(ne["be"]))
  ee = params["edge_enc"]
  ee_w1 = jnp.pad(ee["mlp"][0]["W"], ((0, 5), (0, 0)))
  ee_args = (ee_w1, _b(ee["mlp"][0]["b"]), ee["mlp"][1]["W"],
             _b(ee["mlp"][1]["b"]), ee["mlp"][2]["W"], _b(ee["mlp"][2]["b"]),
             _b(ee["g"]), _b(ee["be"]))

  esplit, eargs, nargs = [], [], []
  for st in params["proc"]:
    ew = st["edge"]["mlp"][0]["W"]
    esplit.append((ew[:LATENT], ew[LATENT:2 * LATENT], ew[2 * LATENT:]))
    em = st["edge"]
    eargs.append((em["mlp"][0]["W"][:LATENT], _b(em["mlp"][0]["b"]),
                  em["mlp"][1]["W"], _b(em["mlp"][1]["b"]),
                  em["mlp"][2]["W"], _b(em["mlp"][2]["b"]),
                  _b(em["g"]), _b(em["be"])))
    nm = st["node"]
    nw = nm["mlp"][0]["W"]
    nargs.append((nw[:LATENT], nw[LATENT:], _b(nm["mlp"][0]["b"]),
                  nm["mlp"][1]["W"], _b(nm["mlp"][1]["b"]),
                  nm["mlp"][2]["W"], _b(nm["mlp"][2]["b"]),
                  _b(nm["g"]), _b(nm["be"])))

  dm = params["dec"]["mlp"]
  d3 = jnp.pad(dm[2]["W"], ((0, 0), (0, 8 - DIM)))
  db3 = _b(jnp.pad(dm[2]["b"], (0, 8 - DIM)))
  dec_args = (dm[0]["W"], _b(dm[0]["b"]), dm[1]["W"], _b(dm[1]["b"]),
              d3, db3)

  nl, tstack = _node_enc_call(
      cp, ty, temb, ne_args + (esplit[0][1], esplit[0][2]))
  pos_tab = jnp.pad(cp[:, 10:12], ((0, 0), (0, LATENT - DIM)))
  posneg = jnp.concatenate([pos_tab, -pos_tab], axis=0)
  gsum = _make_gather_sum()(tstack.reshape(2 * N, LATENT), ialls, iallr)
  reld = _make_gather_sum()(posneg, ialls, iallr)
  el = _edge0_call(gsum.reshape(E, LATENT), reld.reshape(E, LATENT),
                   ee_args, eargs[0])

  for t in range(STEPS):
    if t > 0:
      gsum = _make_gather_sum()(tstack.reshape(2 * N, LATENT), ialls, iallr)
      el = _edge_call(el, gsum.reshape(E, LATENT), eargs[t])
    parts = _make_scatter_add()(el.reshape(NCH_S, CS, LATENT), ircv3, zinit)
    if t < STEPS - 1:
      nl, tstack = _node_call(nl, parts, parts, nargs[t],
                              esplit[t + 1][1], esplit[t + 1][2])
    else:
      out = _node_last_call(nl, parts, parts, cp, nargs[t], dec_args)

  return out[:, :DIM]
